# baseline (speedup 1.0000x reference)
import functools
import math

import jax
import jax.numpy as jnp
from jax import lax
from jax.experimental import pallas as pl
from jax.experimental.pallas import tpu as pltpu
from jax.experimental.pallas import tpu_sc as plsc


def _router_kernel(capacity, dump_slot, xb_ref, wrT_ref, tri_ref,
                   idx_ref, cw_ref, carry_ref):
    E = wrT_ref.shape[0]
    BT = xb_ref.shape[0]

    @pl.when(pl.program_id(0) == 0)
    def _():
        carry_ref[...] = jnp.zeros_like(carry_ref)

    xb = xb_ref[...]
    wrT = wrT_ref[...]
    lt = lax.dot_general(wrT, xb, (((1,), (1,)), ((), ())),
                         preferred_element_type=jnp.float32)
    m = jnp.max(lt, axis=0, keepdims=True)
    gate = 1.0 / jnp.sum(jnp.exp(lt - m), axis=0, keepdims=True)
    eiota = lax.broadcasted_iota(jnp.int32, (E, BT), 0)
    is_max = lt == m
    eidx = jnp.min(jnp.where(is_max, eiota, E), axis=0,
                   keepdims=True)
    oh = (eiota == eidx).astype(jnp.float32)
    excl = lax.dot(oh, tri_ref[...], preferred_element_type=jnp.float32)
    carry = carry_ref[:, 0:1]
    pos = jnp.sum(oh * (excl + carry), axis=0, keepdims=True)
    carry_ref[...] = carry_ref[...] + jnp.sum(oh, axis=1, keepdims=True)
    keep = pos < capacity
    pos_i = pos.astype(jnp.int32)
    slot = jnp.where(keep, eidx * capacity + pos_i, dump_slot)
    idx_ref[...] = slot.reshape(1, 1, BT)
    cw_ref[...] = jnp.where(keep, gate, 0.0).reshape(1, 1, BT)


def _route(xf, W_router, capacity, dump_slot, block_t):
    T, D = xf.shape
    E = W_router.shape[1]
    nb = T // block_t
    tri = (lax.broadcasted_iota(jnp.float32, (block_t, block_t), 0)
           < lax.broadcasted_iota(jnp.float32, (block_t, block_t), 1)
           ).astype(jnp.float32)
    idx3, cw3 = pl.pallas_call(
        functools.partial(_router_kernel, capacity, dump_slot),
        grid=(nb,),
        in_specs=[
            pl.BlockSpec((block_t, D), lambda b: (b, 0)),
            pl.BlockSpec((E, D), lambda b: (0, 0)),
            pl.BlockSpec((block_t, block_t), lambda b: (0, 0)),
        ],
        out_specs=[
            pl.BlockSpec((1, 1, block_t), lambda b: (b, 0, 0)),
            pl.BlockSpec((1, 1, block_t), lambda b: (b, 0, 0)),
        ],
        out_shape=[
            jax.ShapeDtypeStruct((nb, 1, block_t), jnp.int32),
            jax.ShapeDtypeStruct((nb, 1, block_t), jnp.float32),
        ],
        scratch_shapes=[pltpu.VMEM((E, 128), jnp.float32)],
    )(xf, W_router.T, tri)
    return idx3.reshape(T), cw3.reshape(T)


def _dispatch(xf, idx3, nslot, R=128):
    T, D = xf.shape
    NW = 32
    ntok_w = T // NW
    nchunk = ntok_w // R
    mesh = plsc.VectorSubcoreMesh(core_axis_name="c", subcore_axis_name="s")

    @functools.partial(
        pl.kernel, mesh=mesh,
        out_type=jax.ShapeDtypeStruct((nslot, D), jnp.float32),
        scratch_types=[
            pltpu.VMEM((nchunk, R), jnp.int32),
            pltpu.VMEM((R, D), jnp.float32),
            pltpu.SemaphoreType.DMA,
        ],
    )
    def dispatch(x_hbm, idx_hbm, buf_hbm, idx_all, rows_v, sem):
        wid = lax.axis_index("s") * 2 + lax.axis_index("c")
        base = wid * ntok_w
        pltpu.sync_copy(idx_hbm.at[wid], idx_all)

        def body(c, _):
            off = base + c * R
            pltpu.sync_copy(x_hbm.at[pl.ds(off, R)], rows_v)
            pltpu.async_copy(rows_v, buf_hbm.at[idx_all.at[c]], sem).wait()
            return 0

        lax.fori_loop(0, nchunk, body, 0)

    return dispatch(xf, idx3)


def _ffn_kernel(buf_ref, w1_ref, w2_ref, out_ref):
    h = jnp.maximum(
        lax.dot(buf_ref[...], w1_ref[0], preferred_element_type=jnp.float32),
        0.0)
    out_ref[...] = lax.dot(h, w2_ref[0], preferred_element_type=jnp.float32)


def _ffn(buf, W1, W2, capacity):
    nslot, D = buf.shape
    E, _, DFF = W1.shape
    grid = nslot // capacity
    return pl.pallas_call(
        _ffn_kernel,
        grid=(grid,),
        in_specs=[
            pl.BlockSpec((capacity, D), lambda e: (e, 0)),
            pl.BlockSpec((1, D, DFF), lambda e: (jnp.minimum(e, E - 1), 0, 0)),
            pl.BlockSpec((1, DFF, D), lambda e: (jnp.minimum(e, E - 1), 0, 0)),
        ],
        out_specs=pl.BlockSpec((capacity, D), lambda e: (e, 0)),
        out_shape=jax.ShapeDtypeStruct((nslot, D), jnp.float32),
    )(buf, W1, W2)


def _combine(ffn_out, idx3, cw, T, R=128):
    D = ffn_out.shape[1]
    NW = 32
    ntok_w = T // NW
    nchunk = ntok_w // R
    L = 16
    mesh = plsc.VectorSubcoreMesh(core_axis_name="c", subcore_axis_name="s")

    @functools.partial(
        pl.kernel, mesh=mesh,
        out_type=jax.ShapeDtypeStruct((T, D), jnp.float32),
        scratch_types=[
            pltpu.VMEM((nchunk, R), jnp.int32),
            pltpu.VMEM((ntok_w + L,), jnp.float32),
            pltpu.VMEM((R, D), jnp.float32),
            pltpu.SemaphoreType.DMA,
            pltpu.SemaphoreType.DMA,
        ],
    )
    def combine(ffn_hbm, idx_hbm, cw_hbm, y_hbm, idx_all, cw_all,
                rows_v, sg, sst):
        wid = lax.axis_index("s") * 2 + lax.axis_index("c")
        base = wid * ntok_w
        pltpu.sync_copy(idx_hbm.at[wid], idx_all)
        pltpu.sync_copy(cw_hbm.at[pl.ds(base, ntok_w)],
                        cw_all.at[pl.ds(0, ntok_w)])

        def body(c, _):
            coff = c * R
            pltpu.async_copy(ffn_hbm.at[idx_all.at[c]], rows_v, sg).wait()

            def rowbody(j, _):
                cwj = cw_all[pl.ds(coff + j, L)][0]
                for k in range(D // L):
                    sl = rows_v[j, pl.ds(k * L, L)]
                    rows_v[j, pl.ds(k * L, L)] = sl * cwj
                return 0

            lax.fori_loop(0, R, rowbody, 0)
            pltpu.async_copy(rows_v, y_hbm.at[pl.ds(base + coff, R)],
                             sst).wait()
            return 0

        lax.fori_loop(0, nchunk, body, 0)

    return combine(ffn_out, idx3, cw)


def kernel(x, W_router, W1, W2):
    B, S, D = x.shape
    E = W_router.shape[1]
    T = B * S
    capacity = int(math.ceil(T / E))
    dump_slot = E * capacity
    nslot = (E + 1) * capacity

    xf = x.reshape(T, D)
    idx, cw = _route(xf, W_router, capacity, dump_slot, block_t=512)
    R = 128
    NW = 32
    idx3 = idx.reshape(NW, (T // NW) // R, R)
    buf = _dispatch(xf, idx3, nslot, R=R)
    ffn_out = _ffn(buf, W1, W2, capacity)
    y = _combine(ffn_out, idx3, cw, T, R=R)
    return y.reshape(B, S, D)


def _dispatch_lin(xf, idx3, nslot, R=64):
    T, D = xf.shape
    NW = 32
    ntok_w = T // NW
    nchunk = ntok_w // R
    npair = nchunk // 2
    mesh = plsc.VectorSubcoreMesh(core_axis_name="c", subcore_axis_name="s")

    @functools.partial(
        pl.kernel, mesh=mesh,
        out_type=jax.ShapeDtypeStruct((nslot, D), jnp.float32),
        scratch_types=[
            pltpu.VMEM((nchunk, R), jnp.int32),
            pltpu.VMEM((R, D), jnp.float32),
            pltpu.VMEM((R, D), jnp.float32),
            pltpu.SemaphoreType.DMA,
            pltpu.SemaphoreType.DMA,
            pltpu.SemaphoreType.DMA,
            pltpu.SemaphoreType.DMA,
        ],
    )
    def dispatch(x_hbm, idx_hbm, buf_hbm, idx_all, rows0, rows1,
                 in0, in1, sc0, sc1):
        wid = lax.axis_index("s") * 2 + lax.axis_index("c")
        base = wid * ntok_w
        rows = (rows0, rows1)
        sin = (in0, in1)
        ssc = (sc0, sc1)

        def load(c, b):
            off = base + c * R
            pltpu.async_copy(x_hbm.at[pl.ds(off, R)], rows[b], sin[b])

        def store(c, b):
            off = base + c * R
            pltpu.async_copy(rows[b], buf_hbm.at[pl.ds(off, R)], ssc[b])

        def wait_in(b):
            pltpu.make_async_copy(x_hbm.at[pl.ds(0, R)], rows[b],
                                  sin[b]).wait()

        def wait_sc(b):
            pltpu.make_async_copy(x_hbm.at[pl.ds(0, R)], rows[b],
                                  ssc[b]).wait()

        pltpu.sync_copy(idx_hbm.at[wid], idx_all)
        load(0, 0)

        def body(k, _):
            c0 = 2 * k
            wait_in(0)
            store(c0, 0)

            @pl.when(k > 0)
            def _():
                wait_sc(1)

            load(c0 + 1, 1)
            wait_in(1)
            store(c0 + 1, 1)
            wait_sc(0)

            @pl.when(k < npair - 1)
            def _():
                load(c0 + 2, 0)

            return 0

        lax.fori_loop(0, npair, body, 0)
        wait_sc(1)

    return dispatch(xf, idx3)


def _kernel_probe_stage(x, W_router, W1, W2, stage):
    B, S, D = x.shape
    E = W_router.shape[1]
    T = B * S
    capacity = int(math.ceil(T / E))
    dump_slot = E * capacity
    nslot = (E + 1) * capacity
    xf = x.reshape(T, D)
    idx, cw = _route(xf, W_router, capacity, dump_slot, block_t=1024)
    if stage == 1:
        return idx, cw
    R = 64
    NW = 32
    idx3 = idx.reshape(NW, (T // NW) // R, R)
    idx4 = idx.reshape(NW, (T // NW) // 16, 16)
    if stage == 20:
        return _dispatch_lin(xf, idx3, nslot, R=R)
    buf = _dispatch(xf, idx4, nslot, R=R)
    if stage == 2:
        return buf
    ffn_out = _ffn(buf, W1, W2, capacity)
    if stage == 3:
        return ffn_out
    y = _combine(ffn_out, idx3, cw, T, R=R)
    return y.reshape(B, S, D)


# speedup vs baseline: 1.1678x; 1.1678x over previous
import functools
import math

import jax
import jax.numpy as jnp
from jax import lax
from jax.experimental import pallas as pl
from jax.experimental.pallas import tpu as pltpu
from jax.experimental.pallas import tpu_sc as plsc


def _router_kernel(capacity, dump_slot, xb_ref, wrT_ref, tri_ref,
                   idx_ref, gidx_ref, cw_ref, carry_ref):
    E = wrT_ref.shape[0]
    BT = xb_ref.shape[0]

    @pl.when(pl.program_id(0) == 0)
    def _():
        carry_ref[...] = jnp.zeros_like(carry_ref)

    xb = xb_ref[...]
    wrT = wrT_ref[...]
    lt = lax.dot_general(wrT, xb, (((1,), (1,)), ((), ())),
                         preferred_element_type=jnp.float32)
    m = jnp.max(lt, axis=0, keepdims=True)
    gate = 1.0 / jnp.sum(jnp.exp(lt - m), axis=0, keepdims=True)
    eiota = lax.broadcasted_iota(jnp.int32, (E, BT), 0)
    is_max = lt == m
    eidx = jnp.min(jnp.where(is_max, eiota, E), axis=0,
                   keepdims=True)
    oh = (eiota == eidx).astype(jnp.float32)
    excl = lax.dot(oh, tri_ref[...], preferred_element_type=jnp.float32)
    carry = carry_ref[:, 0:1]
    pos = jnp.sum(oh * (excl + carry), axis=0, keepdims=True)
    carry_ref[...] = carry_ref[...] + jnp.sum(oh, axis=1, keepdims=True)
    keep = pos < capacity
    pos_i = pos.astype(jnp.int32)
    slot = jnp.where(keep, eidx * capacity + pos_i, dump_slot)
    gidx = eidx * capacity + jnp.where(keep, pos_i, 0)
    idx_ref[...] = slot.reshape(1, 1, BT)
    gidx_ref[...] = gidx.reshape(1, 1, BT)
    cw_ref[...] = jnp.where(keep, gate, 0.0).reshape(1, 1, BT)


def _route(xf, W_router, capacity, dump_slot, block_t):
    T, D = xf.shape
    E = W_router.shape[1]
    nb = T // block_t
    tri = (lax.broadcasted_iota(jnp.float32, (block_t, block_t), 0)
           < lax.broadcasted_iota(jnp.float32, (block_t, block_t), 1)
           ).astype(jnp.float32)
    idx3, gidx3, cw3 = pl.pallas_call(
        functools.partial(_router_kernel, capacity, dump_slot),
        grid=(nb,),
        in_specs=[
            pl.BlockSpec((block_t, D), lambda b: (b, 0)),
            pl.BlockSpec((E, D), lambda b: (0, 0)),
            pl.BlockSpec((block_t, block_t), lambda b: (0, 0)),
        ],
        out_specs=[
            pl.BlockSpec((1, 1, block_t), lambda b: (b, 0, 0)),
            pl.BlockSpec((1, 1, block_t), lambda b: (b, 0, 0)),
            pl.BlockSpec((1, 1, block_t), lambda b: (b, 0, 0)),
        ],
        out_shape=[
            jax.ShapeDtypeStruct((nb, 1, block_t), jnp.int32),
            jax.ShapeDtypeStruct((nb, 1, block_t), jnp.int32),
            jax.ShapeDtypeStruct((nb, 1, block_t), jnp.float32),
        ],
        scratch_shapes=[pltpu.VMEM((E, 128), jnp.float32)],
    )(xf, W_router.T, tri)
    return idx3.reshape(T), gidx3.reshape(T), cw3.reshape(T)


def _dispatch(xf, idx3, nslot, R=128):
    T, D = xf.shape
    NW = 32
    ntok_w = T // NW
    nchunk = ntok_w // R
    mesh = plsc.VectorSubcoreMesh(core_axis_name="c", subcore_axis_name="s")

    @functools.partial(
        pl.kernel, mesh=mesh,
        out_type=jax.ShapeDtypeStruct((nslot, D), jnp.float32),
        scratch_types=[
            pltpu.VMEM((nchunk, R), jnp.int32),
            pltpu.VMEM((R, D), jnp.float32),
            pltpu.SemaphoreType.DMA,
        ],
    )
    def dispatch(x_hbm, idx_hbm, buf_hbm, idx_all, rows_v, sem):
        wid = lax.axis_index("s") * 2 + lax.axis_index("c")
        base = wid * ntok_w
        pltpu.sync_copy(idx_hbm.at[wid], idx_all)

        def body(c, _):
            off = base + c * R
            pltpu.sync_copy(x_hbm.at[pl.ds(off, R)], rows_v)
            pltpu.async_copy(rows_v, buf_hbm.at[idx_all.at[c]], sem).wait()
            return 0

        lax.fori_loop(0, nchunk, body, 0)

    return dispatch(xf, idx3)


def _ffn_kernel(buf_ref, w1_ref, w2_ref, out_ref):
    h = jnp.maximum(
        lax.dot(buf_ref[...], w1_ref[0], preferred_element_type=jnp.float32),
        0.0)
    out_ref[...] = lax.dot(h, w2_ref[0], preferred_element_type=jnp.float32)


def _ffn(buf, W1, W2, capacity):
    nslot, D = buf.shape
    E, _, DFF = W1.shape
    return pl.pallas_call(
        _ffn_kernel,
        grid=(E,),
        in_specs=[
            pl.BlockSpec((capacity, D), lambda e: (e, 0)),
            pl.BlockSpec((1, D, DFF), lambda e: (e, 0, 0)),
            pl.BlockSpec((1, DFF, D), lambda e: (e, 0, 0)),
        ],
        out_specs=pl.BlockSpec((capacity, D), lambda e: (e, 0)),
        out_shape=jax.ShapeDtypeStruct((E * capacity, D), jnp.float32),
    )(buf, W1, W2)


def _combine(ffn_out, idx3, cw, T, R=128):
    D = ffn_out.shape[1]
    NW = 32
    ntok_w = T // NW
    nchunk = ntok_w // R
    L = 16
    mesh = plsc.VectorSubcoreMesh(core_axis_name="c", subcore_axis_name="s")

    @functools.partial(
        pl.kernel, mesh=mesh,
        out_type=jax.ShapeDtypeStruct((T, D), jnp.float32),
        scratch_types=[
            pltpu.VMEM((nchunk, R), jnp.int32),
            pltpu.VMEM((ntok_w + L,), jnp.float32),
            pltpu.VMEM((R, D), jnp.float32),
            pltpu.SemaphoreType.DMA,
            pltpu.SemaphoreType.DMA,
        ],
    )
    def combine(ffn_hbm, idx_hbm, cw_hbm, y_hbm, idx_all, cw_all,
                rows_v, sg, sst):
        wid = lax.axis_index("s") * 2 + lax.axis_index("c")
        base = wid * ntok_w
        pltpu.sync_copy(idx_hbm.at[wid], idx_all)
        pltpu.sync_copy(cw_hbm.at[pl.ds(base, ntok_w)],
                        cw_all.at[pl.ds(0, ntok_w)])

        def body(c, _):
            coff = c * R
            pltpu.async_copy(ffn_hbm.at[idx_all.at[c]], rows_v, sg).wait()

            def rowbody(j, _):
                cwj = cw_all[pl.ds(coff + j, L)][0]
                for k in range(D // L):
                    sl = rows_v[j, pl.ds(k * L, L)]
                    rows_v[j, pl.ds(k * L, L)] = sl * cwj
                return 0

            lax.fori_loop(0, R, rowbody, 0)
            pltpu.async_copy(rows_v, y_hbm.at[pl.ds(base + coff, R)],
                             sst).wait()
            return 0

        lax.fori_loop(0, nchunk, body, 0)

    return combine(ffn_out, idx3, cw)


def kernel(x, W_router, W1, W2):
    B, S, D = x.shape
    E = W_router.shape[1]
    T = B * S
    capacity = int(math.ceil(T / E))
    dump_slot = E * capacity
    nslot = (E + 1) * capacity

    xf = x.reshape(T, D)
    idx, gidx, cw = _route(xf, W_router, capacity, dump_slot, block_t=1024)
    R = 128
    NW = 32
    idx3 = idx.reshape(NW, (T // NW) // R, R)
    gidx3 = gidx.reshape(NW, (T // NW) // R, R)
    buf = _dispatch(xf, idx3, nslot, R=R)
    ffn_out = _ffn(buf, W1, W2, capacity)
    y = _combine(ffn_out, gidx3, cw, T, R=R)
    return y.reshape(B, S, D)
